# baseline (speedup 1.0000x reference)
import functools

import jax
import jax.numpy as jnp
from jax import lax
from jax.experimental import pallas as pl
from jax.experimental.pallas import tpu as pltpu
from jax.experimental.pallas import tpu_sc as plsc
from jax.scipy.special import erf

_P, _B, _NC, _NBT, _NDOWN, _A = 64, 512, 3, 30, 3, 8192
_NW = 32
_PPW = _P // _NW
_PAIRS_W = _PPW * _B
_NCOMP = 19
_STEPS = _PAIRS_W // 16

@functools.cache
def _make_sc_gather():
    mesh = plsc.VectorSubcoreMesh(core_axis_name="c", subcore_axis_name="s",
                                  num_cores=2, num_subcores=16)
    return functools.partial(
        pl.kernel,
        out_type=jax.ShapeDtypeStruct((_NCOMP * _P * _B,), jnp.float32),
        mesh=mesh,
        compiler_params=pltpu.CompilerParams(needs_layout_passes=False),
        scratch_types=[
            pltpu.VMEM((3, _A // 128, _PPW, 128), jnp.float32),
            pltpu.VMEM((_B // 128, _PPW, 128), jnp.int32),
            pltpu.VMEM((_B // 128, _PPW, 128), jnp.int32),
            pltpu.VMEM((_PAIRS_W * _NC * 2,), jnp.int32),
            pltpu.VMEM((32,), jnp.int32),
            pltpu.VMEM((272,), jnp.int32),
            pltpu.VMEM((_NCOMP * _PAIRS_W,), jnp.float32),
        ],
    )(_sc_gather_body)


def _sc_gather_body(coords_h, bt_h, off_h, inter_h, dconns_h, down_h,
                    out_h, coords_v, bt_v, off_v, inter_v, dconns_v, down_v,
                    out_v):
    c = lax.axis_index("c")
    s = lax.axis_index("s")
    w = s * 2 + c

    pt = lax.shift_right_logical(w, 2)
    ps0 = jnp.bitwise_and(w, 3) * _PPW
    pltpu.sync_copy(coords_h.at[:, pt, :, pl.ds(ps0, _PPW), :], coords_v)
    pltpu.sync_copy(bt_h.at[pt, :, pl.ds(ps0, _PPW), :], bt_v)
    pltpu.sync_copy(off_h.at[pt, :, pl.ds(ps0, _PPW), :], off_v)
    pltpu.sync_copy(inter_h.at[pl.ds(w * (_PAIRS_W * _NC * 2),
                                     _PAIRS_W * _NC * 2)], inter_v)
    pltpu.sync_copy(dconns_h, dconns_v)
    pltpu.sync_copy(down_h, down_v)

    lane = lax.iota(jnp.int32, 16)

    @plsc.parallel_loop(0, _STEPS, unroll=8)
    def step(i):
        base = i * 16
        ploc = lax.shift_right_logical(i, 5)
        btile = jnp.bitwise_and(lax.shift_right_logical(i, 3), 3)
        bl0 = jnp.bitwise_and(i, 7) * 16
        b_in = btile * 128 + bl0 + lane
        bt16 = bt_v[btile, ploc, pl.ds(bl0, 16)]
        dconn = plsc.load_gather(dconns_v, [bt16])
        dconn_c = jnp.clip(dconn, 0, _NC - 1)
        ibase = (((ploc * _NC + dconn_c) * 4 + btile) * 256
                 + bl0 + lane)
        nbrb = plsc.load_gather(inter_v, [ibase])
        nbrc = lax.rem(plsc.load_gather(inter_v, [ibase + 128]),
                       jnp.int32(_NC))
        valid = (dconn >= 0) & (nbrb >= 0) & (b_in < nbrb)
        nbrb_c = jnp.clip(nbrb, 0, _B - 1)
        ntile = lax.shift_right_logical(nbrb_c, 7)
        nlane = jnp.bitwise_and(nbrb_c, 127)
        ploc_v = lane * 0 + ploc
        nbt = plsc.load_gather(bt_v, [ntile, ploc_v, nlane])
        ooff = off_v[btile, ploc, pl.ds(bl0, 16)]
        noff = plsc.load_gather(off_v, [ntile, ploc_v, nlane])
        obase = bt16 * (_NC * _NDOWN) + dconn_c * _NDOWN
        nbase = nbt * (_NC * _NDOWN) + nbrc * _NDOWN
        for k in range(_NDOWN):
            odk = plsc.load_gather(down_v, [obase + k])
            oatom = jnp.clip(ooff + odk, 0, _A - 1)
            oat = lax.shift_right_logical(oatom, 7)
            oal = jnp.bitwise_and(oatom, 127)
            ndk = plsc.load_gather(down_v, [nbase + k])
            natom = jnp.clip(noff + ndk, 0, _A - 1)
            nat = lax.shift_right_logical(natom, 7)
            nal = jnp.bitwise_and(natom, 127)
            for cc in range(3):
                cc_v = lane * 0 + cc
                out_v[pl.ds((k * 3 + cc) * _PAIRS_W + base, 16)] = (
                    plsc.load_gather(coords_v, [cc_v, oat, ploc_v, oal]))
                out_v[pl.ds((9 + k * 3 + cc) * _PAIRS_W + base, 16)] = (
                    plsc.load_gather(coords_v, [cc_v, nat, ploc_v, nal]))
        out_v[pl.ds(18 * _PAIRS_W + base, 16)] = jnp.where(
            valid, jnp.float32(1.0), jnp.float32(0.0))

    for j in range(_NCOMP):
        pltpu.sync_copy(out_v.at[pl.ds(j * _PAIRS_W, _PAIRS_W)],
                        out_h.at[pl.ds(j * (_P * _B) + w * _PAIRS_W,
                                       _PAIRS_W)])


def _sub(a, b):
    return (a[0] - b[0], a[1] - b[1], a[2] - b[2])


def _dot(a, b):
    return a[0] * b[0] + a[1] * b[1] + a[2] * b[2]


def _cross(a, b):
    return (a[1] * b[2] - a[2] * b[1],
            a[2] * b[0] - a[0] * b[2],
            a[0] * b[1] - a[1] * b[0])


def _vnorm(a):
    return jnp.sqrt(_dot(a, a) + 1e-12)


def _acos(x):
    return 2.0 * jnp.arctan2(jnp.sqrt(1.0 - x), jnp.sqrt(1.0 + x))


def _angle(a, b, c):
    v1 = _sub(a, b)
    v2 = _sub(c, b)
    cosang = _dot(v1, v2) / (_vnorm(v1) * _vnorm(v2))
    return _acos(jnp.clip(cosang, -1.0 + 1e-6, 1.0 - 1e-6))


def _dihedral(a, b, c, d):
    b1 = _sub(b, a)
    b2 = _sub(c, b)
    b3 = _sub(d, c)
    n1 = _cross(b1, b2)
    n2 = _cross(b2, b3)
    nb2 = _vnorm(b2)
    b2n = (b2[0] / nb2, b2[1] / nb2, b2[2] / nb2)
    m1 = _cross(n1, b2n)
    x = _dot(n1, n2)
    y = _dot(m1, n2)
    return jnp.arctan2(y, x + 1e-12)


def _tc_body(comp_ref, gp_ref, out_ref):
    g = [gp_ref[i] for i in range(13)]
    SG1 = (comp_ref[0], comp_ref[1], comp_ref[2])
    CB1 = (comp_ref[3], comp_ref[4], comp_ref[5])
    CA1 = (comp_ref[6], comp_ref[7], comp_ref[8])
    SG2 = (comp_ref[9], comp_ref[10], comp_ref[11])
    CB2 = (comp_ref[12], comp_ref[13], comp_ref[14])
    CA2 = (comp_ref[15], comp_ref[16], comp_ref[17])
    valid = comp_ref[18]

    d = _vnorm(_sub(SG1, SG2))
    ang1 = _angle(CB1, SG1, SG2)
    ang2 = _angle(CB2, SG2, SG1)
    dss = _dihedral(CB1, SG1, SG2, CB2)
    chi1 = _dihedral(CA1, CB1, SG1, SG2)
    chi2 = _dihedral(CA2, CB2, SG2, SG1)

    z = (d - (2.02 + g[0])) / (0.35 + g[1])
    f_d = 0.5 * z * z - jnp.log(
        0.5 * (1.0 + erf(g[2] * z / jnp.sqrt(jnp.float32(2.0)))) + 1e-6)
    f_a = (-(g[3] + g[4] * jnp.cos(ang1 - g[5]))
           - (g[3] + g[4] * jnp.cos(ang2 - g[5])))

    def mix(x):
        return -jnp.log(jnp.exp(g[6] + g[7] * jnp.cos(x - g[8]))
                        + jnp.exp(g[9] + g[10] * jnp.cos(x - g[11]))
                        + 1e-9)

    E = (f_d + f_a + mix(dss) + mix(chi1) + mix(chi2)) * (0.5 + g[12])
    E = jnp.where(valid > 0.5, E, 0.0)
    out_ref[...] = jnp.sum(E.reshape(_P, _B // 128, 128), axis=(1, 2))


def kernel(coords, pose_stack_block_coord_offset, pose_stack_block_types,
           pose_stack_inter_block_connections, bt_disulfide_conns,
           bt_atom_downstream_of_conn, global_params):
    coords_f = (coords.astype(jnp.float32)
                .transpose(2, 0, 1)
                .reshape(3, _P // 8, 8, _A // 128, 128)
                .transpose(0, 1, 3, 2, 4))

    def _tiled2d(x):
        return (x.astype(jnp.int32)
                .reshape(_P // 8, 8, _B // 128, 128)
                .transpose(0, 2, 1, 3))

    bt_f = _tiled2d(pose_stack_block_types)
    off_f = _tiled2d(pose_stack_block_coord_offset)

    inter_f = (pose_stack_inter_block_connections.astype(jnp.int32)
               .transpose(0, 2, 3, 1)
               .reshape(_P, _NC, 2, _B // 128, 128)
               .transpose(0, 1, 3, 2, 4)
               .reshape(-1))
    dconns_p = jnp.pad(bt_disulfide_conns.astype(jnp.int32), (0, 32 - _NBT))
    down_p = jnp.pad(
        bt_atom_downstream_of_conn.astype(jnp.int32).reshape(-1),
        (0, 272 - _NBT * _NC * _NDOWN))
    gp = global_params.astype(jnp.float32)

    comp = _make_sc_gather()(coords_f, bt_f, off_f, inter_f, dconns_p,
                             down_p)
    comp3 = comp.reshape(_NCOMP, _P * _B // 128, 128)

    return pl.pallas_call(
        _tc_body,
        out_shape=jax.ShapeDtypeStruct((_P,), jnp.float32),
        in_specs=[pl.BlockSpec(memory_space=pltpu.VMEM),
                  pl.BlockSpec(memory_space=pltpu.SMEM)],
        out_specs=pl.BlockSpec(memory_space=pltpu.VMEM),
    )(comp3, gp)


# speedup vs baseline: 1.0464x; 1.0464x over previous
import functools

import jax
import jax.numpy as jnp
from jax import lax
from jax.experimental import pallas as pl
from jax.experimental.pallas import tpu as pltpu
from jax.experimental.pallas import tpu_sc as plsc
from jax.scipy.special import erf

_P, _B, _NC, _NBT, _NDOWN, _A = 64, 512, 3, 30, 3, 8192
_NW = 32
_PPW = _P // _NW
_PAIRS_W = _PPW * _B
_NCOMP = 19
_STEPS = _PAIRS_W // 16

@functools.cache
def _make_sc_gather():
    mesh = plsc.VectorSubcoreMesh(core_axis_name="c", subcore_axis_name="s",
                                  num_cores=2, num_subcores=16)
    return functools.partial(
        pl.kernel,
        out_type=jax.ShapeDtypeStruct((_NCOMP * _P * _B,), jnp.float32),
        mesh=mesh,
        compiler_params=pltpu.CompilerParams(needs_layout_passes=False),
        scratch_types=[
            pltpu.VMEM((3, _A // 128, _PPW, 128), jnp.float32),
            pltpu.VMEM((_B // 128, _PPW, 128), jnp.int32),
            pltpu.VMEM((_B // 128, _PPW, 128), jnp.int32),
            pltpu.VMEM((_PAIRS_W * _NC * 2,), jnp.int32),
            pltpu.VMEM((32,), jnp.int32),
            pltpu.VMEM((272,), jnp.int32),
            pltpu.VMEM((_NCOMP * _PAIRS_W,), jnp.float32),
        ],
    )(_sc_gather_body)


def _sc_gather_body(coords_h, bt_h, off_h, inter_h, dconns_h, down_h,
                    out_h, coords_v, bt_v, off_v, inter_v, dconns_v, down_v,
                    out_v):
    c = lax.axis_index("c")
    s = lax.axis_index("s")
    w = s * 2 + c

    pt = lax.shift_right_logical(w, 2)
    ps0 = jnp.bitwise_and(w, 3) * _PPW
    pltpu.sync_copy(coords_h.at[:, pt, :, pl.ds(ps0, _PPW), :], coords_v)
    pltpu.sync_copy(bt_h.at[pt, :, pl.ds(ps0, _PPW), :], bt_v)
    pltpu.sync_copy(off_h.at[pt, :, pl.ds(ps0, _PPW), :], off_v)
    pltpu.sync_copy(inter_h.at[pl.ds(w * (_PAIRS_W * _NC * 2),
                                     _PAIRS_W * _NC * 2)], inter_v)
    pltpu.sync_copy(dconns_h, dconns_v.at[pl.ds(0, _NBT)])
    pltpu.sync_copy(down_h, down_v.at[pl.ds(0, _NBT * _NC * _NDOWN)])

    lane = lax.iota(jnp.int32, 16)

    @plsc.parallel_loop(0, _STEPS, unroll=4)
    def step(i):
        base = i * 16
        ploc = lax.shift_right_logical(i, 5)
        btile = jnp.bitwise_and(lax.shift_right_logical(i, 3), 3)
        bl0 = jnp.bitwise_and(i, 7) * 16
        b_in = btile * 128 + bl0 + lane
        bt16 = bt_v[btile, ploc, pl.ds(bl0, 16)]
        dconn = plsc.load_gather(dconns_v, [bt16])
        dconn_c = jnp.clip(dconn, 0, _NC - 1)
        ibase = (((ploc * _NC + dconn_c) * 4 + btile) * 256
                 + bl0 + lane)
        nbrb = plsc.load_gather(inter_v, [ibase])
        nbrc = lax.rem(plsc.load_gather(inter_v, [ibase + 128]),
                       jnp.int32(_NC))
        valid = (dconn >= 0) & (nbrb >= 0) & (b_in < nbrb)
        nbrb_c = jnp.clip(nbrb, 0, _B - 1)
        ntile = lax.shift_right_logical(nbrb_c, 7)
        nlane = jnp.bitwise_and(nbrb_c, 127)
        ploc_v = lane * 0 + ploc
        nbt = plsc.load_gather(bt_v, [ntile, ploc_v, nlane])
        ooff = off_v[btile, ploc, pl.ds(bl0, 16)]
        noff = plsc.load_gather(off_v, [ntile, ploc_v, nlane])
        obase = bt16 * (_NC * _NDOWN) + dconn_c * _NDOWN
        nbase = nbt * (_NC * _NDOWN) + nbrc * _NDOWN
        for k in range(_NDOWN):
            odk = plsc.load_gather(down_v, [obase + k])
            oatom = jnp.clip(ooff + odk, 0, _A - 1)
            oat = lax.shift_right_logical(oatom, 7)
            oal = jnp.bitwise_and(oatom, 127)
            ndk = plsc.load_gather(down_v, [nbase + k])
            natom = jnp.clip(noff + ndk, 0, _A - 1)
            nat = lax.shift_right_logical(natom, 7)
            nal = jnp.bitwise_and(natom, 127)
            for cc in range(3):
                cc_v = lane * 0 + cc
                out_v[pl.ds((k * 3 + cc) * _PAIRS_W + base, 16)] = (
                    plsc.load_gather(coords_v, [cc_v, oat, ploc_v, oal]))
                out_v[pl.ds((9 + k * 3 + cc) * _PAIRS_W + base, 16)] = (
                    plsc.load_gather(coords_v, [cc_v, nat, ploc_v, nal]))
        out_v[pl.ds(18 * _PAIRS_W + base, 16)] = jnp.where(
            valid, jnp.float32(1.0), jnp.float32(0.0))

    for j in range(_NCOMP):
        pltpu.sync_copy(out_v.at[pl.ds(j * _PAIRS_W, _PAIRS_W)],
                        out_h.at[pl.ds(j * (_P * _B) + w * _PAIRS_W,
                                       _PAIRS_W)])


def _sub(a, b):
    return (a[0] - b[0], a[1] - b[1], a[2] - b[2])


def _dot(a, b):
    return a[0] * b[0] + a[1] * b[1] + a[2] * b[2]


def _cross(a, b):
    return (a[1] * b[2] - a[2] * b[1],
            a[2] * b[0] - a[0] * b[2],
            a[0] * b[1] - a[1] * b[0])


def _vnorm(a):
    return jnp.sqrt(_dot(a, a) + 1e-12)


def _acos(x):
    return 2.0 * jnp.arctan2(jnp.sqrt(1.0 - x), jnp.sqrt(1.0 + x))


def _angle(a, b, c):
    v1 = _sub(a, b)
    v2 = _sub(c, b)
    cosang = _dot(v1, v2) / (_vnorm(v1) * _vnorm(v2))
    return _acos(jnp.clip(cosang, -1.0 + 1e-6, 1.0 - 1e-6))


def _dihedral(a, b, c, d):
    b1 = _sub(b, a)
    b2 = _sub(c, b)
    b3 = _sub(d, c)
    n1 = _cross(b1, b2)
    n2 = _cross(b2, b3)
    nb2 = _vnorm(b2)
    b2n = (b2[0] / nb2, b2[1] / nb2, b2[2] / nb2)
    m1 = _cross(n1, b2n)
    x = _dot(n1, n2)
    y = _dot(m1, n2)
    return jnp.arctan2(y, x + 1e-12)


def _tc_body(comp_ref, gp_ref, out_ref):
    g = [gp_ref[i] for i in range(13)]
    SG1 = (comp_ref[0], comp_ref[1], comp_ref[2])
    CB1 = (comp_ref[3], comp_ref[4], comp_ref[5])
    CA1 = (comp_ref[6], comp_ref[7], comp_ref[8])
    SG2 = (comp_ref[9], comp_ref[10], comp_ref[11])
    CB2 = (comp_ref[12], comp_ref[13], comp_ref[14])
    CA2 = (comp_ref[15], comp_ref[16], comp_ref[17])
    valid = comp_ref[18]

    d = _vnorm(_sub(SG1, SG2))
    ang1 = _angle(CB1, SG1, SG2)
    ang2 = _angle(CB2, SG2, SG1)
    dss = _dihedral(CB1, SG1, SG2, CB2)
    chi1 = _dihedral(CA1, CB1, SG1, SG2)
    chi2 = _dihedral(CA2, CB2, SG2, SG1)

    z = (d - (2.02 + g[0])) / (0.35 + g[1])
    f_d = 0.5 * z * z - jnp.log(
        0.5 * (1.0 + erf(g[2] * z / jnp.sqrt(jnp.float32(2.0)))) + 1e-6)
    f_a = (-(g[3] + g[4] * jnp.cos(ang1 - g[5]))
           - (g[3] + g[4] * jnp.cos(ang2 - g[5])))

    def mix(x):
        return -jnp.log(jnp.exp(g[6] + g[7] * jnp.cos(x - g[8]))
                        + jnp.exp(g[9] + g[10] * jnp.cos(x - g[11]))
                        + 1e-9)

    E = (f_d + f_a + mix(dss) + mix(chi1) + mix(chi2)) * (0.5 + g[12])
    E = jnp.where(valid > 0.5, E, 0.0)
    out_ref[...] = jnp.sum(E.reshape(_P, _B // 128, 128), axis=(1, 2))


def kernel(coords, pose_stack_block_coord_offset, pose_stack_block_types,
           pose_stack_inter_block_connections, bt_disulfide_conns,
           bt_atom_downstream_of_conn, global_params):
    coords_f = (coords.astype(jnp.float32)
                .transpose(2, 0, 1)
                .reshape(3, _P // 8, 8, _A // 128, 128)
                .transpose(0, 1, 3, 2, 4))

    def _tiled2d(x):
        return (x.astype(jnp.int32)
                .reshape(_P // 8, 8, _B // 128, 128)
                .transpose(0, 2, 1, 3))

    bt_f = _tiled2d(pose_stack_block_types)
    off_f = _tiled2d(pose_stack_block_coord_offset)

    inter_f = (pose_stack_inter_block_connections.astype(jnp.int32)
               .transpose(0, 2, 3, 1)
               .reshape(_P, _NC, 2, _B // 128, 128)
               .transpose(0, 1, 3, 2, 4)
               .reshape(-1))
    dconns_p = bt_disulfide_conns.astype(jnp.int32)
    down_p = bt_atom_downstream_of_conn.astype(jnp.int32).reshape(-1)
    gp = global_params.astype(jnp.float32)

    comp = _make_sc_gather()(coords_f, bt_f, off_f, inter_f, dconns_p,
                             down_p)
    comp3 = comp.reshape(_NCOMP, _P * _B // 128, 128)

    return pl.pallas_call(
        _tc_body,
        out_shape=jax.ShapeDtypeStruct((_P,), jnp.float32),
        in_specs=[pl.BlockSpec(memory_space=pltpu.VMEM),
                  pl.BlockSpec(memory_space=pltpu.SMEM)],
        out_specs=pl.BlockSpec(memory_space=pltpu.VMEM),
    )(comp3, gp)
